# baseline (speedup 1.0000x reference)
import functools

import jax
import jax.numpy as jnp
from jax import lax
from jax.experimental import pallas as pl
from jax.experimental.pallas import tpu as pltpu
from jax.experimental.pallas import tpu_sc as plsc

_CHUNK = 128


def _gather_call(VL, D, B):
    info = plsc.get_sparse_core_info()
    NC, NS = info.num_cores, info.num_subcores
    NW = NC * NS
    b_per_w = B // NW
    n_chunk = b_per_w // _CHUNK
    grp_per_chunk = _CHUNK // 16
    mesh = plsc.VectorSubcoreMesh(core_axis_name="c", subcore_axis_name="s")

    @functools.partial(
        pl.kernel,
        mesh=mesh,
        out_type=jax.ShapeDtypeStruct((B, D), jnp.float32),
        scratch_types=[
            pltpu.VMEM((b_per_w,), jnp.int32),
            pltpu.VMEM((n_chunk, _CHUNK), jnp.int32),
            pltpu.VMEM((_CHUNK, 128), jnp.float32),
            pltpu.VMEM((b_per_w, D), jnp.float32),
            pltpu.SemaphoreType.DMA,
        ],
        compiler_params=pltpu.CompilerParams(needs_layout_passes=False),
    )
    def k(table_hbm, idx_hbm, out_hbm, idx_v, line_v, lines_v, out_v, sem):
        wid = lax.axis_index("s") * NC + lax.axis_index("c")
        base = wid * b_per_w
        pltpu.sync_copy(idx_hbm.at[pl.ds(base, b_per_w)], idx_v)

        def line_body(g, _):
            iv = idx_v[pl.ds(g * 16, 16)]
            line_v[g // grp_per_chunk, pl.ds((g % grp_per_chunk) * 16, 16)] = (
                lax.shift_right_logical(iv, 3)
            )
            return 0

        lax.fori_loop(0, n_chunk * grp_per_chunk, line_body, 0)

        lane = lax.iota(jnp.int32, 16)

        for c in range(n_chunk):
            pltpu.async_copy(table_hbm.at[line_v.at[c]], lines_v, sem).wait()

            def extract_body(g, _, c=c):
                iv = idx_v[pl.ds(c * _CHUNK + g * 16, 16)]
                off = lax.shift_left(jnp.bitwise_and(iv, 7), 4)
                loc_row = g * 16 + lane
                glb_row = c * _CHUNK + g * 16 + lane
                for j in range(D):
                    valj = plsc.load_gather(lines_v, [loc_row, off + j])
                    plsc.store_scatter(
                        out_v, [glb_row, jnp.full((16,), j, jnp.int32)], valj
                    )
                return 0

            lax.fori_loop(0, grp_per_chunk, extract_body, 0)

        pltpu.sync_copy(out_v, out_hbm.at[pl.ds(base, b_per_w)])

    return k


def kernel(data, channelindex):
    V, D = data.shape
    (B,) = channelindex.shape
    table = data.reshape(V // 8, 128)
    return _gather_call(V // 8, D, B)(table, channelindex.astype(jnp.int32))


# speedup vs baseline: 4.6913x; 4.6913x over previous
import functools

import jax
import jax.numpy as jnp
from jax import lax
from jax.experimental import pallas as pl
from jax.experimental.pallas import tpu as pltpu
from jax.experimental.pallas import tpu_sc as plsc

_K = 16
_HALF = 256


def _gather_call(V, D, B):
    info = plsc.get_sparse_core_info()
    NC, NS = info.num_cores, info.num_subcores
    NW = NC * NS
    b_per_w = B // NW
    n_half = b_per_w // _HALF
    n_chunk = _HALF // _K
    mesh = plsc.VectorSubcoreMesh(core_axis_name="c", subcore_axis_name="s")

    @functools.partial(
        pl.kernel,
        mesh=mesh,
        out_type=jax.ShapeDtypeStruct((B, D), jnp.float32),
        scratch_types=[
            pltpu.VMEM((b_per_w,), jnp.int32),
            pltpu.VMEM((_K, D, 128), jnp.float32),
            pltpu.VMEM((_HALF, D), jnp.float32),
            pltpu.SemaphoreType.DMA,
        ],
        compiler_params=pltpu.CompilerParams(needs_layout_passes=False),
    )
    def k(table_hbm, idx_hbm, out_hbm, idx_v, ring, out_v, sem):
        wid = lax.axis_index("s") * NC + lax.axis_index("c")
        base = wid * b_per_w
        pltpu.sync_copy(idx_hbm.at[pl.ds(base, b_per_w)], idx_v)

        lane = lax.iota(jnp.int32, 16)
        neg_inf = jnp.int32(-2147483648)

        for half in range(n_half):
            def chunk_body(c, _, half=half):
                i0 = half * _HALF + c * _K
                iv16 = idx_v[pl.ds(i0, _K)]
                rs = [
                    jnp.max(jnp.where(lane == j, iv16, neg_inf))
                    for j in range(_K)
                ]
                handles = []
                for j in range(_K):
                    blk = pl.multiple_of(
                        jnp.bitwise_and(rs[j], jnp.int32(~127)), 128
                    )
                    handles.append(
                        pltpu.async_copy(
                            table_hbm.at[:, pl.ds(blk, 128)], ring.at[j], sem
                        )
                    )
                for h in handles:
                    h.wait()
                for j in range(_K):
                    sub = jnp.bitwise_and(rs[j], jnp.int32(127))
                    val = plsc.load_gather(
                        ring.at[j], [lane, jnp.full((16,), sub, jnp.int32)]
                    )
                    row = c * _K + j
                    plsc.store_scatter(
                        out_v, [jnp.full((16,), row, jnp.int32), lane], val
                    )
                return 0

            lax.fori_loop(0, n_chunk, chunk_body, 0)
            pltpu.sync_copy(
                out_v, out_hbm.at[pl.ds(base + half * _HALF, _HALF)]
            )

    return k


def kernel(data, channelindex):
    V, D = data.shape
    (B,) = channelindex.shape
    return _gather_call(V, D, B)(data.T, channelindex.astype(jnp.int32))


# speedup vs baseline: 5.5477x; 1.1826x over previous
import functools

import jax
import jax.numpy as jnp
from jax import lax
from jax.experimental import pallas as pl
from jax.experimental.pallas import tpu as pltpu
from jax.experimental.pallas import tpu_sc as plsc

_K = 16
_HALF = 256


def _gather_call(V, D, B):
    info = plsc.get_sparse_core_info()
    NC, NS = info.num_cores, info.num_subcores
    NW = NC * NS
    b_per_w = B // NW
    n_half = b_per_w // _HALF
    n_chunk = _HALF // _K
    mesh = plsc.VectorSubcoreMesh(core_axis_name="c", subcore_axis_name="s")

    @functools.partial(
        pl.kernel,
        mesh=mesh,
        out_type=jax.ShapeDtypeStruct((B, D), jnp.float32),
        scratch_types=[
            pltpu.VMEM((b_per_w,), jnp.int32),
            pltpu.VMEM((2, _K, D, 128), jnp.float32),
            pltpu.VMEM((_HALF, D), jnp.float32),
            pltpu.SemaphoreType.DMA,
            pltpu.SemaphoreType.DMA,
        ],
        compiler_params=pltpu.CompilerParams(needs_layout_passes=False),
    )
    def k(table_hbm, idx_hbm, out_hbm, idx_v, ring, out_v, sem0, sem1):
        wid = lax.axis_index("s") * NC + lax.axis_index("c")
        base = wid * b_per_w
        pltpu.sync_copy(idx_hbm.at[pl.ds(base, b_per_w)], idx_v)

        lane = lax.iota(jnp.int32, 16)
        neg_inf = jnp.int32(-2147483648)
        sems = [sem0, sem1]
        n_chunks_total = n_half * n_chunk

        def scalarize(ci):
            iv16 = idx_v[pl.ds(ci * _K, _K)]
            return [
                jnp.max(jnp.where(lane == j, iv16, neg_inf))
                for j in range(_K)
            ]

        def fire(ci, bank):
            rs = scalarize(ci)
            for j in range(_K):
                blk = pl.multiple_of(
                    jnp.bitwise_and(rs[j], jnp.int32(~127)), 128
                )
                pltpu.async_copy(
                    table_hbm.at[:, pl.ds(blk, 128)],
                    ring.at[bank, j],
                    sems[bank],
                )

        def drain_extract(ci, lc, bank):
            rs = scalarize(ci)
            for j in range(_K):
                pltpu.make_async_copy(
                    table_hbm.at[:, pl.ds(0, 128)],
                    ring.at[bank, j],
                    sems[bank],
                ).wait()
            for j in range(_K):
                sub = jnp.bitwise_and(rs[j], jnp.int32(127))
                val = plsc.load_gather(
                    ring.at[bank, j], [lane, jnp.full((16,), sub, jnp.int32)]
                )
                row = lc * _K + j
                plsc.store_scatter(
                    out_v, [jnp.full((16,), row, jnp.int32), lane], val
                )

        fire(jnp.int32(0), 0)

        for half in range(n_half):
            def pair_body(p, _, half=half):
                lc0 = 2 * p
                ci0 = jnp.int32(half * n_chunk) + lc0
                fire(ci0 + 1, 1)
                drain_extract(ci0, lc0, 0)

                @pl.when(ci0 + 2 < n_chunks_total)
                def _():
                    fire(ci0 + 2, 0)

                drain_extract(ci0 + 1, lc0 + 1, 1)
                return 0

            lax.fori_loop(0, n_chunk // 2, pair_body, 0)
            pltpu.sync_copy(
                out_v, out_hbm.at[pl.ds(base + half * _HALF, _HALF)]
            )

    return k


def kernel(data, channelindex):
    V, D = data.shape
    (B,) = channelindex.shape
    return _gather_call(V, D, B)(data.T, channelindex.astype(jnp.int32))
